# baseline (speedup 1.0000x reference)
import jax
import jax.numpy as jnp
from jax import lax
from jax.experimental import pallas as pl
from jax.experimental.pallas import tpu as pltpu
from jax.experimental.pallas import tpu_sc as plsc

N = 10000
D = 128
DH = 64
E = 320000
NC = 2
NS = 16
NW = NC * NS
CHUNK = 112
NBUF = 5
NBATCH = 36
K2 = NBUF * NBATCH
E_PAD = NS * K2 * CHUNK
EDEG = E_PAD // NW
NACC = 10112
R_INIT = NACC // NS
R_OUT = R_INIT
RB = 2000


def _vmesh():
    return plsc.VectorSubcoreMesh(core_axis_name="c", subcore_axis_name="s")


def _deg_body(dstf, out, deg_v, dst_v):
    c = lax.axis_index("c")
    s = lax.axis_index("s")
    wid = c * NS + s

    def zero_body(i, carry):
        deg_v[pl.ds(i * 16, 16)] = jnp.zeros((16,), jnp.float32)
        return carry

    lax.fori_loop(0, NACC // 16, zero_body, 0)
    pltpu.sync_copy(dstf.at[wid], dst_v)

    def body(j, carry):
        idx = dst_v[pl.ds(j * 16, 16)]
        plsc.addupdate_scatter(deg_v, [idx], jnp.ones((16,), jnp.float32))
        return carry

    lax.fori_loop(0, EDEG // 16, body, 0)
    pltpu.sync_copy(deg_v, out.at[wid])


def _deg_call(dstf):
    k = pl.kernel(
        _deg_body,
        out_type=jax.ShapeDtypeStruct((NW, NACC), jnp.float32),
        mesh=_vmesh(),
        scratch_types=[
            pltpu.VMEM((NACC,), jnp.float32),
            pltpu.VMEM((EDEG,), jnp.int32),
        ],
        compiler_params=pltpu.CompilerParams(needs_layout_passes=False),
    )
    return k(dstf)


def _agg_body(table, srcp2, dstp, zeros, out, acc,
              is0, is1, is2, is3, id0, id1, id2, id3,
              bufA, bufB, gsA, gsB, ssA, ssB, xs0, xs1, xs2, xs3):
    c = lax.axis_index("c")
    s = lax.axis_index("s")
    pltpu.sync_copy(zeros.at[pl.ds(s * R_INIT, R_INIT)],
                    acc.at[pl.ds(s * R_INIT, R_INIT)])
    plsc.subcore_barrier()

    slots = ((is0, id0, xs0), (is1, id1, xs1),
             (is2, id2, xs2), (is3, id3, xs3))
    grps = ((bufA, gsA, ssA), (bufB, gsB, ssB))

    def load_idx(g, q):
        is_, id_, xs = slots[q]
        pltpu.async_copy(srcp2.at[c, s, pl.ds(g * NBUF, NBUF)], is_, xs)
        pltpu.async_copy(dstp.at[s, pl.ds(g * NBUF, NBUF)], id_, xs)

    def wait_idx(q):
        is_, id_, xs = slots[q]
        pltpu.make_async_copy(srcp2.at[c, s, pl.ds(0, NBUF)], is_, xs).wait()
        pltpu.make_async_copy(dstp.at[s, pl.ds(0, NBUF)], id_, xs).wait()

    def gathers(p, q):
        buf, gs, _ = grps[p]
        is_ = slots[q][0]
        for b in range(NBUF):
            pltpu.async_copy(table.at[is_.at[b]], buf.at[b], gs)

    def scatters(p, q):
        buf, _, ss = grps[p]
        id_ = slots[q][1]
        for b in range(NBUF):
            pltpu.async_copy(buf.at[b], acc.at[id_.at[b]], ss, add=True)

    def drain(p, which):
        buf, gs, ss = grps[p]
        sem = gs if which == "g" else ss
        for _ in range(NBUF):
            pltpu.make_async_copy(table.at[pl.ds(0, CHUNK)], buf.at[0],
                                  sem).wait()

    def step(g, p, q, drain_prev, has_next, has_next2):
        if has_next2:
            load_idx(g + 2, (q + 2) % 4)
        drain(p, "g")
        scatters(p, q)
        if drain_prev:
            drain(1 - p, "s")
        if has_next:
            wait_idx((q + 1) % 4)
            gathers(1 - p, (q + 1) % 4)

    load_idx(0, 0)
    load_idx(1, 1)
    wait_idx(0)
    gathers(0, 0)
    step(0, 0, 0, False, True, True)
    step(1, 1, 1, True, True, True)

    def quad(g4, carry):
        g = 2 + 4 * g4
        step(g + 0, 0, 2, True, True, True)
        step(g + 1, 1, 3, True, True, True)
        step(g + 2, 0, 0, True, True, True)
        step(g + 3, 1, 1, True, True, True)
        return carry

    lax.fori_loop(0, (NBATCH - 4) // 4, quad, 0)
    step(NBATCH - 2, 0, 2, True, True, False)
    step(NBATCH - 1, 1, 3, True, False, False)
    drain(1, "s")

    plsc.subcore_barrier()
    pltpu.sync_copy(acc.at[pl.ds(s * R_OUT, R_OUT)],
                    out.at[c, pl.ds(s * R_OUT, R_OUT)])


def _agg_call(table, srcp2, dstp, zeros):
    k = pl.kernel(
        _agg_body,
        out_type=jax.ShapeDtypeStruct((NC, NACC, DH), jnp.float32),
        mesh=_vmesh(),
        scratch_types=(
            [pltpu.VMEM_SHARED((NACC, DH), jnp.float32)]
            + [pltpu.VMEM((NBUF, CHUNK), jnp.int32) for _ in range(8)]
            + [pltpu.VMEM((NBUF, CHUNK, DH), jnp.float32) for _ in range(2)]
            + [pltpu.SemaphoreType.DMA for _ in range(8)]
        ),
        compiler_params=pltpu.CompilerParams(use_tc_tiling_on_sc=False),
    )
    return k(table, srcp2, dstp, zeros)


def _s_block(deg_blk):
    return lax.rsqrt(1.0 + jnp.sum(deg_blk, axis=1))[:, None]


_DEG_SPEC = pl.BlockSpec((RB, NW), lambda i: (i, 0))


def _mm_raw(x, W):
    def body(x_ref, w_ref, o_ref):
        o_ref[...] = jnp.dot(x_ref[...], w_ref[...],
                             preferred_element_type=jnp.float32)

    return pl.pallas_call(
        body,
        grid=(N // RB,),
        in_specs=[
            pl.BlockSpec((RB, D), lambda i: (i, 0)),
            pl.BlockSpec((D, D), lambda i: (0, 0)),
        ],
        out_specs=pl.BlockSpec((RB, D), lambda i: (i, 0)),
        out_shape=jax.ShapeDtypeStruct((N, D), jnp.float32),
    )(x, W)


def _scale_split(degp, h):
    def body(d_ref, h_ref, o_ref):
        s = _s_block(d_ref[...])
        res = s * h_ref[...]
        o_ref[0] = res[:, :DH]
        o_ref[1] = res[:, DH:]

    return pl.pallas_call(
        body,
        grid=(N // RB,),
        in_specs=[
            _DEG_SPEC,
            pl.BlockSpec((RB, D), lambda i: (i, 0)),
        ],
        out_specs=pl.BlockSpec((2, RB, DH), lambda i: (0, i, 0)),
        out_shape=jax.ShapeDtypeStruct((2, N, DH), jnp.float32),
    )(degp, h)


def _merge_mm(degp, hps, p, b, W):
    def body(d_ref, h_ref, p_ref, b_ref, w_ref, o_ref):
        s = _s_block(d_ref[...])
        h = jnp.concatenate([h_ref[0], h_ref[1]], axis=1)
        agg = jnp.concatenate([p_ref[0], p_ref[1]], axis=1)
        t = jnp.maximum(s * (h + agg) + b_ref[...], 0.0)
        res = jnp.dot(s * t, w_ref[...], preferred_element_type=jnp.float32)
        o_ref[0] = res[:, :DH]
        o_ref[1] = res[:, DH:]

    return pl.pallas_call(
        body,
        grid=(N // RB,),
        in_specs=[
            _DEG_SPEC,
            pl.BlockSpec((2, RB, DH), lambda i: (0, i, 0)),
            pl.BlockSpec((2, RB, DH), lambda i: (0, i, 0)),
            pl.BlockSpec((1, D), lambda i: (0, 0)),
            pl.BlockSpec((D, D), lambda i: (0, 0)),
        ],
        out_specs=pl.BlockSpec((2, RB, DH), lambda i: (0, i, 0)),
        out_shape=jax.ShapeDtypeStruct((2, N, DH), jnp.float32),
    )(degp, hps, p, b, W)


def _merge_out(degp, hps, p, b):
    def body(d_ref, h_ref, p_ref, b_ref, o_ref):
        s = _s_block(d_ref[...])
        h = jnp.concatenate([h_ref[0], h_ref[1]], axis=1)
        agg = jnp.concatenate([p_ref[0], p_ref[1]], axis=1)
        o_ref[...] = jnp.maximum(s * (h + agg) + b_ref[...], 0.0)

    return pl.pallas_call(
        body,
        grid=(N // RB,),
        in_specs=[
            _DEG_SPEC,
            pl.BlockSpec((2, RB, DH), lambda i: (0, i, 0)),
            pl.BlockSpec((2, RB, DH), lambda i: (0, i, 0)),
            pl.BlockSpec((1, D), lambda i: (0, 0)),
        ],
        out_specs=pl.BlockSpec((RB, D), lambda i: (i, 0)),
        out_shape=jax.ShapeDtypeStruct((N, D), jnp.float32),
    )(degp, hps, p, b)


def kernel(x, W1, b1, W2, b2, edge_index):
    ei = edge_index.astype(jnp.int32)
    src = ei[0]
    dst = ei[1]
    pad = E_PAD - E
    srcp = jnp.concatenate(
        [src, jnp.zeros((pad,), jnp.int32)]).reshape(NS, K2, CHUNK)
    srcp2 = jnp.stack([srcp, srcp + N])
    dstp = jnp.concatenate(
        [dst, jnp.full((pad,), N, jnp.int32)]).reshape(NS, K2, CHUNK)
    zeros = jnp.zeros((NACC, DH), jnp.float32)

    h1 = _mm_raw(x, W1)
    degp = _deg_call(dstp.reshape(NW, EDEG))[:, :N].T

    hps1 = _scale_split(degp, h1)
    p = _agg_call(hps1.reshape(2 * N, DH), srcp2, dstp, zeros)
    hps2 = _merge_mm(degp, hps1, p, b1.reshape(1, D), W2)
    q = _agg_call(hps2.reshape(2 * N, DH), srcp2, dstp, zeros)
    return _merge_out(degp, hps2, q, b2.reshape(1, D))


# speedup vs baseline: 1.0113x; 1.0113x over previous
import jax
import jax.numpy as jnp
from jax import lax
from jax.experimental import pallas as pl
from jax.experimental.pallas import tpu as pltpu
from jax.experimental.pallas import tpu_sc as plsc

N = 10000
D = 128
DH = 64
E = 320000
NC = 2
NS = 16
NW = NC * NS
CHUNK = 112
NBUF = 5
NBATCH = 36
K2 = NBUF * NBATCH
E_PAD = NS * K2 * CHUNK
EDEG = E_PAD // NW
NACC = 10112
R_INIT = NACC // NS
R_OUT = R_INIT
RB = 2000


def _vmesh():
    return plsc.VectorSubcoreMesh(core_axis_name="c", subcore_axis_name="s")


def _deg_body(dstf, out, deg_v, dst_v):
    c = lax.axis_index("c")
    s = lax.axis_index("s")
    wid = c * NS + s

    def zero_body(i, carry):
        deg_v[pl.ds(i * 16, 16)] = jnp.zeros((16,), jnp.float32)
        return carry

    lax.fori_loop(0, NACC // 16, zero_body, 0)
    pltpu.sync_copy(dstf.at[wid], dst_v)

    def body(j, carry):
        idx = dst_v[pl.ds(j * 16, 16)]
        plsc.addupdate_scatter(deg_v, [idx], jnp.ones((16,), jnp.float32))
        return carry

    lax.fori_loop(0, EDEG // 16, body, 0)
    pltpu.sync_copy(deg_v, out.at[wid])


def _deg_call(dstf):
    k = pl.kernel(
        _deg_body,
        out_type=jax.ShapeDtypeStruct((NW, NACC), jnp.float32),
        mesh=_vmesh(),
        scratch_types=[
            pltpu.VMEM((NACC,), jnp.float32),
            pltpu.VMEM((EDEG,), jnp.int32),
        ],
        compiler_params=pltpu.CompilerParams(needs_layout_passes=False),
    )
    return k(dstf)


_R_LAST = N - (NS - 1) * R_INIT


def _agg_body(table, srcp2, dstp, out, acc,
              is0, is1, is2, is3, id0, id1, id2, id3,
              bufA, bufB, gsA, gsB, ssA, ssB, xs0, xs1, xs2, xs3):
    c = lax.axis_index("c")
    s = lax.axis_index("s")

    @pl.when(s < NS - 1)
    def _():
        pltpu.sync_copy(table.at[pl.ds(c * N + s * R_INIT, R_INIT)],
                        acc.at[pl.ds(s * R_INIT, R_INIT)])

    @pl.when(s == NS - 1)
    def _():
        pltpu.sync_copy(table.at[pl.ds(c * N + (NS - 1) * R_INIT, _R_LAST)],
                        acc.at[pl.ds((NS - 1) * R_INIT, _R_LAST)])

    plsc.subcore_barrier()

    slots = ((is0, id0, xs0), (is1, id1, xs1),
             (is2, id2, xs2), (is3, id3, xs3))
    grps = ((bufA, gsA, ssA), (bufB, gsB, ssB))

    def load_idx(g, q):
        is_, id_, xs = slots[q]
        pltpu.async_copy(srcp2.at[c, s, pl.ds(g * NBUF, NBUF)], is_, xs)
        pltpu.async_copy(dstp.at[s, pl.ds(g * NBUF, NBUF)], id_, xs)

    def wait_idx(q):
        is_, id_, xs = slots[q]
        pltpu.make_async_copy(srcp2.at[c, s, pl.ds(0, NBUF)], is_, xs).wait()
        pltpu.make_async_copy(dstp.at[s, pl.ds(0, NBUF)], id_, xs).wait()

    def gathers(p, q):
        buf, gs, _ = grps[p]
        is_ = slots[q][0]
        for b in range(NBUF):
            pltpu.async_copy(table.at[is_.at[b]], buf.at[b], gs)

    def scatters(p, q):
        buf, _, ss = grps[p]
        id_ = slots[q][1]
        for b in range(NBUF):
            pltpu.async_copy(buf.at[b], acc.at[id_.at[b]], ss, add=True)

    def drain(p, which):
        buf, gs, ss = grps[p]
        sem = gs if which == "g" else ss
        for _ in range(NBUF):
            pltpu.make_async_copy(table.at[pl.ds(0, CHUNK)], buf.at[0],
                                  sem).wait()

    def step(g, p, q, drain_prev, has_next, has_next2):
        if has_next2:
            load_idx(g + 2, (q + 2) % 4)
        drain(p, "g")
        scatters(p, q)
        if drain_prev:
            drain(1 - p, "s")
        if has_next:
            wait_idx((q + 1) % 4)
            gathers(1 - p, (q + 1) % 4)

    load_idx(0, 0)
    load_idx(1, 1)
    wait_idx(0)
    gathers(0, 0)
    step(0, 0, 0, False, True, True)
    step(1, 1, 1, True, True, True)

    def quad(g4, carry):
        g = 2 + 4 * g4
        step(g + 0, 0, 2, True, True, True)
        step(g + 1, 1, 3, True, True, True)
        step(g + 2, 0, 0, True, True, True)
        step(g + 3, 1, 1, True, True, True)
        return carry

    lax.fori_loop(0, (NBATCH - 4) // 4, quad, 0)
    step(NBATCH - 2, 0, 2, True, True, False)
    step(NBATCH - 1, 1, 3, True, False, False)
    drain(1, "s")

    plsc.subcore_barrier()
    pltpu.sync_copy(acc.at[pl.ds(s * R_OUT, R_OUT)],
                    out.at[c, pl.ds(s * R_OUT, R_OUT)])


def _agg_call(table, srcp2, dstp):
    k = pl.kernel(
        _agg_body,
        out_type=jax.ShapeDtypeStruct((NC, NACC, DH), jnp.float32),
        mesh=_vmesh(),
        scratch_types=(
            [pltpu.VMEM_SHARED((NACC, DH), jnp.float32)]
            + [pltpu.VMEM((NBUF, CHUNK), jnp.int32) for _ in range(8)]
            + [pltpu.VMEM((NBUF, CHUNK, DH), jnp.float32) for _ in range(2)]
            + [pltpu.SemaphoreType.DMA for _ in range(8)]
        ),
        compiler_params=pltpu.CompilerParams(use_tc_tiling_on_sc=False),
    )
    return k(table, srcp2, dstp)


def _s_block(deg_blk):
    return lax.rsqrt(1.0 + jnp.sum(deg_blk, axis=1))[:, None]


_DEG_SPEC = pl.BlockSpec((RB, NW), lambda i: (i, 0))


def _mm_scale(degp, x, W):
    def body(d_ref, x_ref, w_ref, o_ref):
        s = _s_block(d_ref[...])
        res = jnp.dot(s * x_ref[...], w_ref[...],
                      preferred_element_type=jnp.float32)
        o_ref[0] = res[:, :DH]
        o_ref[1] = res[:, DH:]

    return pl.pallas_call(
        body,
        grid=(N // RB,),
        in_specs=[
            _DEG_SPEC,
            pl.BlockSpec((RB, D), lambda i: (i, 0)),
            pl.BlockSpec((D, D), lambda i: (0, 0)),
        ],
        out_specs=pl.BlockSpec((2, RB, DH), lambda i: (0, i, 0)),
        out_shape=jax.ShapeDtypeStruct((2, N, DH), jnp.float32),
    )(degp, x, W)


def _merge_mm(degp, p, b, W):
    def body(d_ref, p_ref, b_ref, w_ref, o_ref):
        s = _s_block(d_ref[...])
        agg = jnp.concatenate([p_ref[0], p_ref[1]], axis=1)
        t = jnp.maximum(s * agg + b_ref[...], 0.0)
        res = jnp.dot(s * t, w_ref[...], preferred_element_type=jnp.float32)
        o_ref[0] = res[:, :DH]
        o_ref[1] = res[:, DH:]

    return pl.pallas_call(
        body,
        grid=(N // RB,),
        in_specs=[
            _DEG_SPEC,
            pl.BlockSpec((2, RB, DH), lambda i: (0, i, 0)),
            pl.BlockSpec((1, D), lambda i: (0, 0)),
            pl.BlockSpec((D, D), lambda i: (0, 0)),
        ],
        out_specs=pl.BlockSpec((2, RB, DH), lambda i: (0, i, 0)),
        out_shape=jax.ShapeDtypeStruct((2, N, DH), jnp.float32),
    )(degp, p, b, W)


def _merge_out(degp, p, b):
    def body(d_ref, p_ref, b_ref, o_ref):
        s = _s_block(d_ref[...])
        agg = jnp.concatenate([p_ref[0], p_ref[1]], axis=1)
        o_ref[...] = jnp.maximum(s * agg + b_ref[...], 0.0)

    return pl.pallas_call(
        body,
        grid=(N // RB,),
        in_specs=[
            _DEG_SPEC,
            pl.BlockSpec((2, RB, DH), lambda i: (0, i, 0)),
            pl.BlockSpec((1, D), lambda i: (0, 0)),
        ],
        out_specs=pl.BlockSpec((RB, D), lambda i: (i, 0)),
        out_shape=jax.ShapeDtypeStruct((N, D), jnp.float32),
    )(degp, p, b)


def kernel(x, W1, b1, W2, b2, edge_index):
    ei = edge_index.astype(jnp.int32)
    src = ei[0]
    dst = ei[1]
    pad = E_PAD - E
    srcp = jnp.concatenate(
        [src, jnp.zeros((pad,), jnp.int32)]).reshape(NS, K2, CHUNK)
    srcp2 = jnp.stack([srcp, srcp + N])
    dstp = jnp.concatenate(
        [dst, jnp.full((pad,), N, jnp.int32)]).reshape(NS, K2, CHUNK)

    degp = _deg_call(dstp.reshape(NW, EDEG))[:, :N].T

    hps1 = _mm_scale(degp, x, W1)
    p = _agg_call(hps1.reshape(2 * N, DH), srcp2, dstp)
    hps2 = _merge_mm(degp, p, b1.reshape(1, D), W2)
    q = _agg_call(hps2.reshape(2 * N, DH), srcp2, dstp)
    return _merge_out(degp, q, b2.reshape(1, D))


# speedup vs baseline: 1.0268x; 1.0154x over previous
import jax
import jax.numpy as jnp
from jax import lax
from jax.experimental import pallas as pl
from jax.experimental.pallas import tpu as pltpu
from jax.experimental.pallas import tpu_sc as plsc

N = 10000
D = 128
DH = 64
E = 320000
NC = 2
NS = 16
NW = NC * NS
CHUNK = 112
NBUF = 5
NBATCH = 36
K2 = NBUF * NBATCH
E_PAD = NS * K2 * CHUNK
EDEG = E_PAD // NW
NACC = 10112
R_INIT = NACC // NS
R_OUT = R_INIT
RB = 2000


def _vmesh():
    return plsc.VectorSubcoreMesh(core_axis_name="c", subcore_axis_name="s")


def _deg_body(dstf, out, deg_v, dst_v):
    c = lax.axis_index("c")
    s = lax.axis_index("s")
    wid = c * NS + s

    def zero_body(i, carry):
        deg_v[pl.ds(i * 16, 16)] = jnp.zeros((16,), jnp.float32)
        return carry

    lax.fori_loop(0, NACC // 16, zero_body, 0)
    pltpu.sync_copy(dstf.at[wid], dst_v)

    def body(j, carry):
        idx = dst_v[pl.ds(j * 16, 16)]
        plsc.addupdate_scatter(deg_v, [idx], jnp.ones((16,), jnp.float32))
        return carry

    lax.fori_loop(0, EDEG // 16, body, 0)
    pltpu.sync_copy(deg_v, out.at[wid])


def _deg_call(dstf):
    k = pl.kernel(
        _deg_body,
        out_type=jax.ShapeDtypeStruct((NW, NACC), jnp.float32),
        mesh=_vmesh(),
        scratch_types=[
            pltpu.VMEM((NACC,), jnp.float32),
            pltpu.VMEM((EDEG,), jnp.int32),
        ],
        compiler_params=pltpu.CompilerParams(needs_layout_passes=False),
    )
    return k(dstf)


_R_LAST = N - (NS - 1) * R_INIT


def _agg_body(table, srcp, dstp, out, acc,
              is0, is1, is2, is3, id0, id1, id2, id3,
              bufA, bufB, gsA, gsB, ssA, ssB, xs0, xs1, xs2, xs3):
    c = lax.axis_index("c")
    s = lax.axis_index("s")

    @pl.when(s < NS - 1)
    def _():
        pltpu.sync_copy(table.at[c, pl.ds(s * R_INIT, R_INIT)],
                        acc.at[pl.ds(s * R_INIT, R_INIT)])

    @pl.when(s == NS - 1)
    def _():
        pltpu.sync_copy(table.at[c, pl.ds((NS - 1) * R_INIT, _R_LAST)],
                        acc.at[pl.ds((NS - 1) * R_INIT, _R_LAST)])

    plsc.subcore_barrier()

    slots = ((is0, id0, xs0), (is1, id1, xs1),
             (is2, id2, xs2), (is3, id3, xs3))
    grps = ((bufA, gsA, ssA), (bufB, gsB, ssB))

    def load_idx(g, q):
        is_, id_, xs = slots[q]
        pltpu.async_copy(srcp.at[s, pl.ds(g * NBUF, NBUF)], is_, xs)
        pltpu.async_copy(dstp.at[s, pl.ds(g * NBUF, NBUF)], id_, xs)

    def wait_idx(q):
        is_, id_, xs = slots[q]
        pltpu.make_async_copy(srcp.at[s, pl.ds(0, NBUF)], is_, xs).wait()
        pltpu.make_async_copy(dstp.at[s, pl.ds(0, NBUF)], id_, xs).wait()

    def gathers(p, q):
        buf, gs, _ = grps[p]
        is_ = slots[q][0]
        for b in range(NBUF):
            pltpu.async_copy(table.at[c].at[is_.at[b]], buf.at[b], gs)

    def scatters(p, q):
        buf, _, ss = grps[p]
        id_ = slots[q][1]
        for b in range(NBUF):
            pltpu.async_copy(buf.at[b], acc.at[id_.at[b]], ss, add=True)

    def drain(p, which):
        buf, gs, ss = grps[p]
        sem = gs if which == "g" else ss
        for _ in range(NBUF):
            pltpu.make_async_copy(table.at[0, pl.ds(0, CHUNK)], buf.at[0],
                                  sem).wait()

    def step(g, p, q, drain_prev, has_next, has_next2):
        if has_next2:
            load_idx(g + 2, (q + 2) % 4)
        drain(p, "g")
        scatters(p, q)
        if drain_prev:
            drain(1 - p, "s")
        if has_next:
            wait_idx((q + 1) % 4)
            gathers(1 - p, (q + 1) % 4)

    load_idx(0, 0)
    load_idx(1, 1)
    wait_idx(0)
    gathers(0, 0)
    step(0, 0, 0, False, True, True)
    step(1, 1, 1, True, True, True)

    def quad(g4, carry):
        g = 2 + 4 * g4
        step(g + 0, 0, 2, True, True, True)
        step(g + 1, 1, 3, True, True, True)
        step(g + 2, 0, 0, True, True, True)
        step(g + 3, 1, 1, True, True, True)
        return carry

    lax.fori_loop(0, (NBATCH - 4) // 4, quad, 0)
    step(NBATCH - 2, 0, 2, True, True, False)
    step(NBATCH - 1, 1, 3, True, False, False)
    drain(1, "s")

    plsc.subcore_barrier()
    pltpu.sync_copy(acc.at[pl.ds(s * R_OUT, R_OUT)],
                    out.at[c, pl.ds(s * R_OUT, R_OUT)])


def _agg_call(table, srcp, dstp):
    k = pl.kernel(
        _agg_body,
        out_type=jax.ShapeDtypeStruct((NC, NACC, DH), jnp.float32),
        mesh=_vmesh(),
        scratch_types=(
            [pltpu.VMEM_SHARED((NACC, DH), jnp.float32)]
            + [pltpu.VMEM((NBUF, CHUNK), jnp.int32) for _ in range(8)]
            + [pltpu.VMEM((NBUF, CHUNK, DH), jnp.float32) for _ in range(2)]
            + [pltpu.SemaphoreType.DMA for _ in range(8)]
        ),
        compiler_params=pltpu.CompilerParams(use_tc_tiling_on_sc=False),
    )
    return k(table, srcp, dstp)


def _s_block(deg_blk):
    return lax.rsqrt(1.0 + jnp.sum(deg_blk, axis=1))[:, None]


_DEG_SPEC = pl.BlockSpec((RB, NW), lambda i: (i, 0))


def _mm_scale(degp, x, W):
    def body(d_ref, x_ref, w_ref, o_ref):
        s = _s_block(d_ref[...])
        res = jnp.dot(s * x_ref[...], w_ref[...],
                      preferred_element_type=jnp.float32)
        o_ref[0] = res[:, :DH]
        o_ref[1] = res[:, DH:]

    return pl.pallas_call(
        body,
        grid=(N // RB,),
        in_specs=[
            _DEG_SPEC,
            pl.BlockSpec((RB, D), lambda i: (i, 0)),
            pl.BlockSpec((D, D), lambda i: (0, 0)),
        ],
        out_specs=pl.BlockSpec((2, RB, DH), lambda i: (0, i, 0)),
        out_shape=jax.ShapeDtypeStruct((2, N, DH), jnp.float32),
    )(degp, x, W)


def _merge_mm(degp, p, b, W):
    def body(d_ref, p_ref, b_ref, w_ref, o_ref):
        s = _s_block(d_ref[...])
        agg = jnp.concatenate([p_ref[0], p_ref[1]], axis=1)
        t = jnp.maximum(s * agg + b_ref[...], 0.0)
        res = jnp.dot(s * t, w_ref[...], preferred_element_type=jnp.float32)
        o_ref[0] = res[:, :DH]
        o_ref[1] = res[:, DH:]

    return pl.pallas_call(
        body,
        grid=(N // RB,),
        in_specs=[
            _DEG_SPEC,
            pl.BlockSpec((2, RB, DH), lambda i: (0, i, 0)),
            pl.BlockSpec((1, D), lambda i: (0, 0)),
            pl.BlockSpec((D, D), lambda i: (0, 0)),
        ],
        out_specs=pl.BlockSpec((2, RB, DH), lambda i: (0, i, 0)),
        out_shape=jax.ShapeDtypeStruct((2, N, DH), jnp.float32),
    )(degp, p, b, W)


def _merge_out(degp, p, b):
    def body(d_ref, p_ref, b_ref, o_ref):
        s = _s_block(d_ref[...])
        agg = jnp.concatenate([p_ref[0], p_ref[1]], axis=1)
        o_ref[...] = jnp.maximum(s * agg + b_ref[...], 0.0)

    return pl.pallas_call(
        body,
        grid=(N // RB,),
        in_specs=[
            _DEG_SPEC,
            pl.BlockSpec((2, RB, DH), lambda i: (0, i, 0)),
            pl.BlockSpec((1, D), lambda i: (0, 0)),
        ],
        out_specs=pl.BlockSpec((RB, D), lambda i: (i, 0)),
        out_shape=jax.ShapeDtypeStruct((N, D), jnp.float32),
    )(degp, p, b)


def kernel(x, W1, b1, W2, b2, edge_index):
    ei = edge_index.astype(jnp.int32)
    src = ei[0]
    dst = ei[1]
    pad = E_PAD - E
    srcp = jnp.concatenate(
        [src, jnp.zeros((pad,), jnp.int32)]).reshape(NS, K2, CHUNK)
    dstp = jnp.concatenate(
        [dst, jnp.full((pad,), N, jnp.int32)]).reshape(NS, K2, CHUNK)

    degp = _deg_call(dstp.reshape(NW, EDEG))[:, :N].T

    hps1 = _mm_scale(degp, x, W1)
    p = _agg_call(hps1, srcp, dstp)
    hps2 = _merge_mm(degp, p, b1.reshape(1, D), W2)
    q = _agg_call(hps2, srcp, dstp)
    return _merge_out(degp, q, b2.reshape(1, D))
